# baseline (speedup 1.0000x reference)
import functools

import jax
import jax.numpy as jnp
from jax import lax
from jax.experimental import pallas as pl
from jax.experimental.pallas import tpu as pltpu
from jax.experimental.pallas import tpu_sc as plsc

_NW = 32
_V = 512
_NSC = 65536
_NB = 8192


def _pred_masks(no_b, tgt_b):
    n0 = no_b[0:1, :]
    n1 = no_b[1:2, :]
    n2 = no_b[2:3, :]
    p0 = (n0 >= n1) & (n0 >= n2)
    p1 = jnp.logical_not(p0) & (n1 >= n2)
    p2 = jnp.logical_not(p0 | p1)
    km = ((tgt_b == 1) & p1).astype(jnp.float32)
    tm = ((tgt_b == 2) & p2).astype(jnp.float32)
    tw = ((tgt_b == 2) & jnp.logical_not(p2)).astype(jnp.float32)
    return km, tm, tw


def _sc_p1_body(no_hbm, tg_hbm, f_hbm, vec_out, cnt_out,
                f_buf, no_buf, tg_buf, acc_buf, cnt_buf, *, batch, fd):
    wid = lax.axis_index("s") * 2 + lax.axis_index("c")
    span = _NSC // _NW
    base = wid * span
    zeros16 = jnp.zeros((16,), jnp.float32)
    for r in range(2 * batch):
        for c in range(fd):
            acc_buf[r, c, :] = zeros16
    for b in range(batch):
        for j in range(3):
            cnt_buf[b, j, :] = zeros16

    for b in range(batch):
        def chunk_body(ci, _, b=b):
            v0 = base + ci * _V
            pltpu.sync_copy(f_hbm.at[b, :, pl.ds(v0, _V)], f_buf)
            pltpu.sync_copy(no_hbm.at[b, :, pl.ds(v0, _V)], no_buf)
            pltpu.sync_copy(tg_hbm.at[b, 0, pl.ds(v0, _V)], tg_buf)

            def group_body(g, _, b=b):
                sl = pl.ds(g * 16, 16)
                n0 = no_buf[0, sl]
                n1 = no_buf[1, sl]
                n2 = no_buf[2, sl]
                t16 = tg_buf[sl]
                ones = jnp.full((16,), 1.0, jnp.float32)
                zeros = jnp.zeros((16,), jnp.float32)
                a01 = jnp.where(n0 >= n1, ones, zeros)
                a02 = jnp.where(n0 >= n2, ones, zeros)
                a12 = jnp.where(n1 >= n2, ones, zeros)
                p0f = a01 * a02
                p1f = (1.0 - p0f) * a12
                p2f = (1.0 - p0f) * (1.0 - a12)
                tgt1 = jnp.where(t16 == 1, ones, zeros)
                tgt2 = jnp.where(t16 == 2, ones, zeros)
                km = tgt1 * p1f
                tm = tgt2 * p2f
                tw = tgt2 * (1.0 - p2f)
                plsc.addupdate(cnt_buf.at[b, 0], tm)
                plsc.addupdate(cnt_buf.at[b, 1], tw)
                plsc.addupdate(cnt_buf.at[b, 2], km)
                for c in range(fd):
                    f16 = f_buf[c, sl]
                    plsc.addupdate(acc_buf.at[b, c], f16 * km)
                    plsc.addupdate(acc_buf.at[batch + b, c], f16 * tm)
                return 0

            lax.fori_loop(0, _V // 16, group_body, 0)
            return 0

        lax.fori_loop(0, span // _V, chunk_body, 0)

    pltpu.sync_copy(acc_buf, vec_out.at[wid])
    pltpu.sync_copy(cnt_buf, cnt_out.at[wid])


def _tc_p1_kernel(no_ref, tg_ref, f_ref, vec_ref, cnt_ref, *, batch):
    i = pl.program_id(0)

    @pl.when(i == 0)
    def _init():
        vec_ref[...] = jnp.zeros_like(vec_ref)
        cnt_ref[...] = jnp.zeros_like(cnt_ref)

    lane8 = jax.lax.broadcasted_iota(jnp.int32, (1, 8), 1)
    for b in range(batch):
        km, tm, tw = _pred_masks(no_ref[b], tg_ref[b])
        f = f_ref[b]
        mk = jnp.concatenate([km, tm], axis=0)
        r = jax.lax.dot_general(
            mk, f, (((1,), (1,)), ((), ())),
            precision=jax.lax.Precision.HIGHEST,
            preferred_element_type=jnp.float32)
        vec_ref[b:b + 1, :] += r[0:1, :]
        vec_ref[batch + b:batch + b + 1, :] += r[1:2, :]
        row = (jnp.sum(tm) * (lane8 == 0) + jnp.sum(tw) * (lane8 == 1)
               + jnp.sum(km) * (lane8 == 2))
        cnt_ref[b:b + 1, :] += row


def _tc_p2_kernel(no_ref, tg_ref, f_ref, vec_ref, cnt_ref, dq_ref, out_ref,
                  proto_ref, tvn_ref, w_ref, acc_ref, *, batch, n_total, q):
    i = pl.program_id(0)
    nblocks = pl.num_programs(0)
    inv_n = 1.0 / n_total

    @pl.when(i == 0)
    def _prologue():
        kvm = vec_ref[0:batch, :] * inv_n
        pad = jnp.zeros((16 - q - batch, kvm.shape[1]), jnp.float32)
        proto = jnp.concatenate([dq_ref[...], kvm, pad], axis=0)
        nrm = jnp.sqrt(jnp.sum(proto * proto, axis=1, keepdims=True)) + 1e-8
        proto_ref[...] = proto / nrm
        tvm = vec_ref[batch:2 * batch, :] * inv_n
        tnrm = jnp.sqrt(jnp.sum(tvm * tvm, axis=1, keepdims=True)) + 1e-8
        tvn_ref[...] = tvm / tnrm
        ka0 = cnt_ref[0, 2] > 0.0
        ka1 = cnt_ref[1, 2] > 0.0
        r16 = jax.lax.broadcasted_iota(jnp.int32, (16, 8), 0)
        c16 = jax.lax.broadcasted_iota(jnp.int32, (16, 8), 1)
        w = ((r16 < q) | ((r16 == q) & ka0)
             | ((r16 == q + 1) & (c16 >= 1) & ka1)).astype(jnp.float32)
        w_ref[...] = w
        for j in range(2 * batch):
            acc_ref[j] = 0.0

    for b in range(batch):
        _, _, tw = _pred_masks(no_ref[b], tg_ref[b])
        f = f_ref[b]
        sq = jnp.sum(f * f, axis=0, keepdims=True)
        rn = 1.0 / (jnp.sqrt(sq) + 1e-8)
        dots = jnp.dot(proto_ref[...], f,
                       precision=jax.lax.Precision.HIGHEST,
                       preferred_element_type=jnp.float32)
        e = jnp.exp(dots * rn) * tw
        colsum = jnp.sum(e, axis=1, keepdims=True)
        expk_b = jnp.sum(colsum * w_ref[:, b:b + 1])
        svec = jnp.dot(tvn_ref[b:b + 1, :], f,
                       precision=jax.lax.Precision.HIGHEST,
                       preferred_element_type=jnp.float32)
        s_b = jnp.sum(svec * rn * tw)
        acc_ref[b] += s_b
        acc_ref[batch + b] += expk_b

    @pl.when(i == nblocks - 1)
    def _epilogue():
        et = jnp.float32(0.0)
        ek = jnp.float32(0.0)
        any_c = False
        for b in range(batch):
            c_b = (cnt_ref[b, 0] > 0.0) & (cnt_ref[b, 1] > 0.0)
            et = et + jnp.where(c_b, jnp.exp(acc_ref[b]), 0.0)
            ek = ek + jnp.where(c_b, acc_ref[batch + b], 0.0)
            any_c = c_b | any_c
        denom = jnp.where(any_c, ek, 1.0)
        loss = jnp.where(any_c, (-1.0 / batch) * jnp.log(et / denom), 0.0)
        out_ref[0, 0] = loss


@jax.jit
def _run(net_output, feature, target, kidney_deque):
    b, c, d, h, w = net_output.shape
    fd = feature.shape[1]
    q = kidney_deque.shape[0]
    n_total = d * h * w

    no = net_output.reshape(b, c, n_total)
    f = feature.reshape(b, fd, n_total)
    tg = target.reshape(b, 1, n_total)

    mesh = plsc.VectorSubcoreMesh(
        core_axis_name="c", subcore_axis_name="s",
        num_cores=2, num_subcores=16)
    sc_p1 = pl.kernel(
        functools.partial(_sc_p1_body, batch=b, fd=fd),
        out_type=(
            jax.ShapeDtypeStruct((_NW, 2 * b, fd, 16), jnp.float32),
            jax.ShapeDtypeStruct((_NW, b, 3, 16), jnp.float32),
        ),
        mesh=mesh,
        scratch_types=[
            pltpu.VMEM((fd, _V), jnp.float32),
            pltpu.VMEM((3, _V), jnp.float32),
            pltpu.VMEM((_V,), jnp.int32),
            pltpu.VMEM((2 * b, fd, 16), jnp.float32),
            pltpu.VMEM((b, 3, 16), jnp.float32),
        ],
    )
    sc_vec, sc_cnt = sc_p1(no, tg, f)

    n_tc = n_total - _NSC
    off = _NSC // _NB
    tc_vec, tc_cnt = pl.pallas_call(
        functools.partial(_tc_p1_kernel, batch=b),
        grid=(n_tc // _NB,),
        in_specs=[
            pl.BlockSpec((b, c, _NB), lambda i: (0, 0, i + off)),
            pl.BlockSpec((b, 1, _NB), lambda i: (0, 0, i + off)),
            pl.BlockSpec((b, fd, _NB), lambda i: (0, 0, i + off)),
        ],
        out_specs=[
            pl.BlockSpec((2 * b, fd), lambda i: (0, 0)),
            pl.BlockSpec((b, 8), lambda i: (0, 0)),
        ],
        out_shape=[
            jax.ShapeDtypeStruct((2 * b, fd), jnp.float32),
            jax.ShapeDtypeStruct((b, 8), jnp.float32),
        ],
    )(no, tg, f)

    vec = tc_vec + jnp.sum(sc_vec, axis=(0, 3))
    cnt = tc_cnt[:, :3] + jnp.sum(sc_cnt, axis=(0, 3))

    loss = pl.pallas_call(
        functools.partial(_tc_p2_kernel, batch=b, n_total=n_total, q=q),
        grid=(n_total // _NB,),
        in_specs=[
            pl.BlockSpec((b, c, _NB), lambda i: (0, 0, i)),
            pl.BlockSpec((b, 1, _NB), lambda i: (0, 0, i)),
            pl.BlockSpec((b, fd, _NB), lambda i: (0, 0, i)),
            pl.BlockSpec((2 * b, fd), lambda i: (0, 0)),
            pl.BlockSpec(memory_space=pltpu.SMEM),
            pl.BlockSpec((q, fd), lambda i: (0, 0)),
        ],
        out_specs=pl.BlockSpec(memory_space=pltpu.SMEM),
        out_shape=jax.ShapeDtypeStruct((1, 1), jnp.float32),
        scratch_shapes=[
            pltpu.VMEM((16, fd), jnp.float32),
            pltpu.VMEM((b, fd), jnp.float32),
            pltpu.VMEM((16, 8), jnp.float32),
            pltpu.SMEM((2 * b,), jnp.float32),
        ],
    )(no, tg, f, vec, cnt, kidney_deque)

    return loss[0, 0]


def kernel(net_output, feature, target, kidney_deque, background_deque):
    del background_deque
    return _run(net_output, feature, target, kidney_deque)


# speedup vs baseline: 1.4844x; 1.4844x over previous
import functools

import jax
import jax.numpy as jnp
from jax.experimental import pallas as pl
from jax.experimental.pallas import tpu as pltpu


def _pred_masks(no_b, tgt_b):
    n0 = no_b[0:1, :]
    n1 = no_b[1:2, :]
    n2 = no_b[2:3, :]
    p0 = (n0 >= n1) & (n0 >= n2)
    p1 = jnp.logical_not(p0) & (n1 >= n2)
    p2 = jnp.logical_not(p0 | p1)
    km = ((tgt_b == 1) & p1).astype(jnp.float32)
    tm = ((tgt_b == 2) & p2).astype(jnp.float32)
    tw = ((tgt_b == 2) & jnp.logical_not(p2)).astype(jnp.float32)
    return km, tm, tw


def _fused_kernel(no_ref, tg_ref, f_ref, dq_ref, out_ref,
                  vec_ref, cnt_ref, proto_ref, tvn_ref, w_ref, acc_ref,
                  *, batch, n_total, q):
    ph = pl.program_id(0)
    i = pl.program_id(1)
    nblocks = pl.num_programs(1)
    inv_n = 1.0 / n_total

    @pl.when((ph == 0) & (i == 0))
    def _init():
        vec_ref[...] = jnp.zeros_like(vec_ref)
        for b in range(batch):
            for j in range(3):
                cnt_ref[b, j] = 0.0

    @pl.when(ph == 0)
    def _pass1():
        for b in range(batch):
            km, tm, tw = _pred_masks(no_ref[b], tg_ref[b])
            f = f_ref[b]
            mk = jnp.concatenate([km, tm], axis=0)
            r = jax.lax.dot_general(
                mk, f, (((1,), (1,)), ((), ())),
                precision=jax.lax.Precision.HIGHEST,
                preferred_element_type=jnp.float32)
            vec_ref[b:b + 1, :] += r[0:1, :]
            vec_ref[batch + b:batch + b + 1, :] += r[1:2, :]
            cnt_ref[b, 0] += jnp.sum(tm)
            cnt_ref[b, 1] += jnp.sum(tw)
            cnt_ref[b, 2] += jnp.sum(km)

    @pl.when((ph == 1) & (i == 0))
    def _prologue():
        kvm = vec_ref[0:batch, :] * inv_n
        pad = jnp.zeros((16 - q - batch, kvm.shape[1]), jnp.float32)
        proto = jnp.concatenate([dq_ref[...], kvm, pad], axis=0)
        nrm = jnp.sqrt(jnp.sum(proto * proto, axis=1, keepdims=True)) + 1e-8
        proto_ref[...] = proto / nrm
        tvm = vec_ref[batch:2 * batch, :] * inv_n
        tnrm = jnp.sqrt(jnp.sum(tvm * tvm, axis=1, keepdims=True)) + 1e-8
        tvn_ref[...] = tvm / tnrm
        ka0 = cnt_ref[0, 2] > 0.0
        ka1 = cnt_ref[1, 2] > 0.0
        r16 = jax.lax.broadcasted_iota(jnp.int32, (16, 8), 0)
        c16 = jax.lax.broadcasted_iota(jnp.int32, (16, 8), 1)
        w = ((r16 < q) | ((r16 == q) & ka0)
             | ((r16 == q + 1) & (c16 >= 1) & ka1)).astype(jnp.float32)
        w_ref[...] = w
        for j in range(2 * batch):
            acc_ref[j] = 0.0

    @pl.when(ph == 1)
    def _pass2():
        for b in range(batch):
            _, _, tw = _pred_masks(no_ref[b], tg_ref[b])
            f = f_ref[b]
            sq = jnp.sum(f * f, axis=0, keepdims=True)
            rn = 1.0 / (jnp.sqrt(sq) + 1e-8)
            dots = jnp.dot(proto_ref[...], f,
                           precision=jax.lax.Precision.HIGHEST,
                           preferred_element_type=jnp.float32)
            e = jnp.exp(dots * rn) * tw
            colsum = jnp.sum(e, axis=1, keepdims=True)
            expk_b = jnp.sum(colsum * w_ref[:, b:b + 1])
            svec = jnp.dot(tvn_ref[b:b + 1, :], f,
                           precision=jax.lax.Precision.HIGHEST,
                           preferred_element_type=jnp.float32)
            s_b = jnp.sum(svec * rn * tw)
            acc_ref[b] += s_b
            acc_ref[batch + b] += expk_b

    @pl.when((ph == 1) & (i == nblocks - 1))
    def _epilogue():
        et = jnp.float32(0.0)
        ek = jnp.float32(0.0)
        any_c = False
        for b in range(batch):
            c_b = (cnt_ref[b, 0] > 0.0) & (cnt_ref[b, 1] > 0.0)
            et = et + jnp.where(c_b, jnp.exp(acc_ref[b]), 0.0)
            ek = ek + jnp.where(c_b, acc_ref[batch + b], 0.0)
            any_c = c_b | any_c
        denom = jnp.where(any_c, ek, 1.0)
        loss = jnp.where(any_c, (-1.0 / batch) * jnp.log(et / denom), 0.0)
        out_ref[0, 0] = loss


@jax.jit
def _run(net_output, feature, target, kidney_deque):
    b, c, d, h, w = net_output.shape
    fd = feature.shape[1]
    q = kidney_deque.shape[0]
    n_total = d * h * w
    nb = 16384
    while n_total % nb != 0:
        nb //= 2
    nblocks = n_total // nb

    no = net_output.reshape(b, c, n_total)
    f = feature.reshape(b, fd, n_total)
    tg = target.reshape(b, 1, n_total)

    loss = pl.pallas_call(
        functools.partial(_fused_kernel, batch=b, n_total=n_total, q=q),
        grid=(2, nblocks),
        in_specs=[
            pl.BlockSpec((b, c, nb), lambda p, i: (0, 0, i)),
            pl.BlockSpec((b, 1, nb), lambda p, i: (0, 0, i)),
            pl.BlockSpec((b, fd, nb), lambda p, i: (0, 0, i)),
            pl.BlockSpec((q, fd), lambda p, i: (0, 0)),
        ],
        out_specs=pl.BlockSpec(memory_space=pltpu.SMEM),
        out_shape=jax.ShapeDtypeStruct((1, 1), jnp.float32),
        scratch_shapes=[
            pltpu.VMEM((2 * b, fd), jnp.float32),
            pltpu.SMEM((b, 3), jnp.float32),
            pltpu.VMEM((16, fd), jnp.float32),
            pltpu.VMEM((b, fd), jnp.float32),
            pltpu.VMEM((16, 8), jnp.float32),
            pltpu.SMEM((2 * b,), jnp.float32),
        ],
    )(no, tg, f, kidney_deque)

    return loss[0, 0]


def kernel(net_output, feature, target, kidney_deque, background_deque):
    del background_deque
    return _run(net_output, feature, target, kidney_deque)


# speedup vs baseline: 1.6063x; 1.0821x over previous
import functools

import jax
import jax.numpy as jnp
from jax.experimental import pallas as pl
from jax.experimental.pallas import tpu as pltpu


def _pred_masks(no_b, tgt_b):
    n0 = no_b[0:1, :]
    n1 = no_b[1:2, :]
    n2 = no_b[2:3, :]
    p0 = (n0 >= n1) & (n0 >= n2)
    p1 = jnp.logical_not(p0) & (n1 >= n2)
    p2 = jnp.logical_not(p0 | p1)
    km = ((tgt_b == 1) & p1).astype(jnp.float32)
    tm = ((tgt_b == 2) & p2).astype(jnp.float32)
    tw = ((tgt_b == 2) & jnp.logical_not(p2)).astype(jnp.float32)
    return km, tm, tw


def _fused_kernel(no_ref, tg_ref, f_ref, dq_ref, out_ref,
                  vec_ref, cnt_ref, proto_ref, tvn_ref, w_ref, acc_ref,
                  *, batch, n_total, q):
    ph = pl.program_id(0)
    i = pl.program_id(1)
    nblocks = pl.num_programs(1)
    inv_n = 1.0 / n_total

    @pl.when((ph == 0) & (i == 0))
    def _init():
        vec_ref[...] = jnp.zeros_like(vec_ref)
        for b in range(batch):
            for j in range(3):
                cnt_ref[b, j] = 0.0

    @pl.when(ph == 0)
    def _pass1():
        for b in range(batch):
            km, tm, tw = _pred_masks(no_ref[b], tg_ref[b])
            f = f_ref[b]
            mk = jnp.concatenate([km, tm], axis=0)
            r = jax.lax.dot_general(
                mk, f, (((1,), (1,)), ((), ())),
                precision=jax.lax.Precision.HIGHEST,
                preferred_element_type=jnp.float32)
            vec_ref[b:b + 1, :] += r[0:1, :]
            vec_ref[batch + b:batch + b + 1, :] += r[1:2, :]
            cnt_ref[b, 0] += jnp.sum(tm)
            cnt_ref[b, 1] += jnp.sum(tw)
            cnt_ref[b, 2] += jnp.sum(km)

    @pl.when((ph == 1) & (i == 0))
    def _prologue():
        kvm = vec_ref[0:batch, :] * inv_n
        pad = jnp.zeros((16 - q - batch, kvm.shape[1]), jnp.float32)
        proto = jnp.concatenate([dq_ref[...], kvm, pad], axis=0)
        nrm = jnp.sqrt(jnp.sum(proto * proto, axis=1, keepdims=True)) + 1e-8
        proto_ref[...] = proto / nrm
        tvm = vec_ref[batch:2 * batch, :] * inv_n
        tnrm = jnp.sqrt(jnp.sum(tvm * tvm, axis=1, keepdims=True)) + 1e-8
        tvn_ref[...] = tvm / tnrm
        ka0 = cnt_ref[0, 2] > 0.0
        ka1 = cnt_ref[1, 2] > 0.0
        r16 = jax.lax.broadcasted_iota(jnp.int32, (16, 8), 0)
        c16 = jax.lax.broadcasted_iota(jnp.int32, (16, 8), 1)
        w = ((r16 < q) | ((r16 == q) & ka0)
             | ((r16 == q + 1) & (c16 >= 1) & ka1)).astype(jnp.float32)
        w_ref[...] = w
        for j in range(2 * batch):
            acc_ref[j] = 0.0

    @pl.when(ph == 1)
    def _pass2():
        for b in range(batch):
            _, _, tw = _pred_masks(no_ref[b], tg_ref[b])
            f = f_ref[b]
            sq = jnp.sum(f * f, axis=0, keepdims=True)
            rn = 1.0 / (jnp.sqrt(sq) + 1e-8)
            dots = jnp.dot(proto_ref[...], f,
                           preferred_element_type=jnp.float32)
            e = jnp.exp(dots * rn) * tw
            colsum = jnp.sum(e, axis=1, keepdims=True)
            expk_b = jnp.sum(colsum * w_ref[:, b:b + 1])
            svec = jnp.dot(tvn_ref[b:b + 1, :], f,
                           precision=jax.lax.Precision.HIGHEST,
                           preferred_element_type=jnp.float32)
            s_b = jnp.sum(svec * rn * tw)
            acc_ref[b] += s_b
            acc_ref[batch + b] += expk_b

    @pl.when((ph == 1) & (i == nblocks - 1))
    def _epilogue():
        et = jnp.float32(0.0)
        ek = jnp.float32(0.0)
        any_c = False
        for b in range(batch):
            c_b = (cnt_ref[b, 0] > 0.0) & (cnt_ref[b, 1] > 0.0)
            et = et + jnp.where(c_b, jnp.exp(acc_ref[b]), 0.0)
            ek = ek + jnp.where(c_b, acc_ref[batch + b], 0.0)
            any_c = c_b | any_c
        denom = jnp.where(any_c, ek, 1.0)
        loss = jnp.where(any_c, (-1.0 / batch) * jnp.log(et / denom), 0.0)
        out_ref[0, 0] = loss


@jax.jit
def _run(net_output, feature, target, kidney_deque):
    b, c, d, h, w = net_output.shape
    fd = feature.shape[1]
    q = kidney_deque.shape[0]
    n_total = d * h * w
    nb = 16384
    while n_total % nb != 0:
        nb //= 2
    nblocks = n_total // nb

    no = net_output.reshape(b, c, n_total)
    f = feature.reshape(b, fd, n_total)
    tg = target.reshape(b, 1, n_total)

    loss = pl.pallas_call(
        functools.partial(_fused_kernel, batch=b, n_total=n_total, q=q),
        grid=(2, nblocks),
        in_specs=[
            pl.BlockSpec((b, c, nb), lambda p, i: (0, 0, i)),
            pl.BlockSpec((b, 1, nb), lambda p, i: (0, 0, i)),
            pl.BlockSpec((b, fd, nb), lambda p, i: (0, 0, i)),
            pl.BlockSpec((q, fd), lambda p, i: (0, 0)),
        ],
        out_specs=pl.BlockSpec(memory_space=pltpu.SMEM),
        out_shape=jax.ShapeDtypeStruct((1, 1), jnp.float32),
        scratch_shapes=[
            pltpu.VMEM((2 * b, fd), jnp.float32),
            pltpu.SMEM((b, 3), jnp.float32),
            pltpu.VMEM((16, fd), jnp.float32),
            pltpu.VMEM((b, fd), jnp.float32),
            pltpu.VMEM((16, 8), jnp.float32),
            pltpu.SMEM((2 * b,), jnp.float32),
        ],
    )(no, tg, f, kidney_deque)

    return loss[0, 0]


def kernel(net_output, feature, target, kidney_deque, background_deque):
    del background_deque
    return _run(net_output, feature, target, kidney_deque)


# speedup vs baseline: 1.8840x; 1.1729x over previous
import functools

import jax
import jax.numpy as jnp
from jax.experimental import pallas as pl
from jax.experimental.pallas import tpu as pltpu


def _pred_masks(no_b, tgt_b):
    n0 = no_b[0:1, :]
    n1 = no_b[1:2, :]
    n2 = no_b[2:3, :]
    p0 = (n0 >= n1) & (n0 >= n2)
    p1 = jnp.logical_not(p0) & (n1 >= n2)
    p2 = jnp.logical_not(p0 | p1)
    km = ((tgt_b == 1) & p1).astype(jnp.float32)
    tm = ((tgt_b == 2) & p2).astype(jnp.float32)
    tw = ((tgt_b == 2) & jnp.logical_not(p2)).astype(jnp.float32)
    return km, tm, tw


def _fused_kernel(no_ref, tg_ref, f_ref, dq_ref, out_ref,
                  vec_ref, cnt_ref, proto_ref, cols_ref, acc_ref,
                  *, batch, n_total, q):
    ph = pl.program_id(0)
    i = pl.program_id(1)
    nblocks = pl.num_programs(1)
    inv_n = 1.0 / n_total

    @pl.when((ph == 0) & (i == 0))
    def _init():
        vec_ref[...] = jnp.zeros_like(vec_ref)
        for b in range(batch):
            for j in range(3):
                cnt_ref[b, j] = 0.0

    @pl.when(ph == 0)
    def _pass1():
        for b in range(batch):
            km, tm, tw = _pred_masks(no_ref[b], tg_ref[b])
            f = f_ref[b]
            vec_ref[:, b:b + 1] += jnp.sum(f * km, axis=1, keepdims=True)
            vec_ref[:, batch + b:batch + b + 1] += (
                jnp.sum(f * tm, axis=1, keepdims=True))
            cnt_ref[b, 0] += jnp.sum(tm)
            cnt_ref[b, 1] += jnp.sum(tw)
            cnt_ref[b, 2] += jnp.sum(km)

    @pl.when((ph == 1) & (i == 0))
    def _prologue():
        cols = vec_ref[...] * inv_n
        cnorm = jnp.sqrt(jnp.sum(cols * cols, axis=0, keepdims=True)) + 1e-8
        cols_ref[...] = cols / cnorm
        dq = dq_ref[...]
        dnorm = jnp.sqrt(jnp.sum(dq * dq, axis=1, keepdims=True)) + 1e-8
        proto_ref[...] = dq / dnorm
        for j in range(2 * batch):
            acc_ref[j] = 0.0

    @pl.when(ph == 1)
    def _pass2():
        ka0 = cnt_ref[0, 2] > 0.0
        ka1 = cnt_ref[1, 2] > 0.0
        for b in range(batch):
            _, _, tw = _pred_masks(no_ref[b], tg_ref[b])
            f = f_ref[b]
            sq = jnp.sum(f * f, axis=0, keepdims=True)
            rn = 1.0 / (jnp.sqrt(sq) + 1e-8)
            dots = jnp.dot(proto_ref[...], f,
                           preferred_element_type=jnp.float32)
            ek = jnp.sum(jnp.exp(dots * rn) * tw)
            ck0 = jnp.sum(f * cols_ref[:, 0:1], axis=0, keepdims=True)
            ek0 = jnp.sum(jnp.exp(ck0 * rn) * tw)
            ek = ek + jnp.where(ka0, ek0, 0.0)
            if b >= 1:
                ck1 = jnp.sum(f * cols_ref[:, 1:2], axis=0, keepdims=True)
                ek1 = jnp.sum(jnp.exp(ck1 * rn) * tw)
                ek = ek + jnp.where(ka1, ek1, 0.0)
            sv = jnp.sum(f * cols_ref[:, batch + b:batch + b + 1],
                         axis=0, keepdims=True)
            s_b = jnp.sum(sv * rn * tw)
            acc_ref[b] += s_b
            acc_ref[batch + b] += ek

    @pl.when((ph == 1) & (i == nblocks - 1))
    def _epilogue():
        et = jnp.float32(0.0)
        ek = jnp.float32(0.0)
        any_c = False
        for b in range(batch):
            c_b = (cnt_ref[b, 0] > 0.0) & (cnt_ref[b, 1] > 0.0)
            et = et + jnp.where(c_b, jnp.exp(acc_ref[b]), 0.0)
            ek = ek + jnp.where(c_b, acc_ref[batch + b], 0.0)
            any_c = c_b | any_c
        denom = jnp.where(any_c, ek, 1.0)
        loss = jnp.where(any_c, (-1.0 / batch) * jnp.log(et / denom), 0.0)
        out_ref[0, 0] = loss


@jax.jit
def _run(net_output, feature, target, kidney_deque):
    b, c, d, h, w = net_output.shape
    fd = feature.shape[1]
    q = kidney_deque.shape[0]
    n_total = d * h * w
    nb = 16384
    while n_total % nb != 0:
        nb //= 2
    nblocks = n_total // nb

    no = net_output.reshape(b, c, n_total)
    f = feature.reshape(b, fd, n_total)
    tg = target.reshape(b, 1, n_total)

    loss = pl.pallas_call(
        functools.partial(_fused_kernel, batch=b, n_total=n_total, q=q),
        grid=(2, nblocks),
        in_specs=[
            pl.BlockSpec((b, c, nb), lambda p, i: (0, 0, i)),
            pl.BlockSpec((b, 1, nb), lambda p, i: (0, 0, i)),
            pl.BlockSpec((b, fd, nb), lambda p, i: (0, 0, i)),
            pl.BlockSpec((q, fd), lambda p, i: (0, 0)),
        ],
        out_specs=pl.BlockSpec(memory_space=pltpu.SMEM),
        out_shape=jax.ShapeDtypeStruct((1, 1), jnp.float32),
        scratch_shapes=[
            pltpu.VMEM((fd, 2 * b * 2), jnp.float32),
            pltpu.SMEM((b, 3), jnp.float32),
            pltpu.VMEM((q, fd), jnp.float32),
            pltpu.VMEM((fd, 2 * b * 2), jnp.float32),
            pltpu.SMEM((2 * b,), jnp.float32),
        ],
    )(no, tg, f, kidney_deque)

    return loss[0, 0]


def kernel(net_output, feature, target, kidney_deque, background_deque):
    del background_deque
    return _run(net_output, feature, target, kidney_deque)


# speedup vs baseline: 1.9547x; 1.0375x over previous
import functools

import jax
import jax.numpy as jnp
from jax.experimental import pallas as pl
from jax.experimental.pallas import tpu as pltpu


def _pred_masks(no_b, tgt_b):
    n0 = no_b[0:1, :]
    n1 = no_b[1:2, :]
    n2 = no_b[2:3, :]
    p0 = (n0 >= n1) & (n0 >= n2)
    p1 = jnp.logical_not(p0) & (n1 >= n2)
    p2 = jnp.logical_not(p0 | p1)
    km = ((tgt_b == 1) & p1).astype(jnp.float32)
    tm = ((tgt_b == 2) & p2).astype(jnp.float32)
    tw = ((tgt_b == 2) & jnp.logical_not(p2)).astype(jnp.float32)
    return km, tm, tw


def _fused_kernel(no_ref, tg_ref, f_ref, dq_ref, out_ref,
                  vec_ref, cnt_ref, proto_ref, cols_ref, acc_ref,
                  *, batch, n_total, q):
    ph = pl.program_id(0)
    i = pl.program_id(1)
    nblocks = pl.num_programs(1)
    inv_n = 1.0 / n_total

    @pl.when((ph == 0) & (i == 0))
    def _init():
        vec_ref[...] = jnp.zeros_like(vec_ref)
        for b in range(batch):
            for j in range(3):
                cnt_ref[b, j] = 0.0

    @pl.when(ph == 0)
    def _pass1():
        for b in range(batch):
            km, tm, tw = _pred_masks(no_ref[b], tg_ref[b])
            f = f_ref[b]
            vec_ref[:, b:b + 1] += jnp.sum(f * km, axis=1, keepdims=True)
            vec_ref[:, batch + b:batch + b + 1] += (
                jnp.sum(f * tm, axis=1, keepdims=True))
            cnt_ref[b, 0] += jnp.sum(tm)
            cnt_ref[b, 1] += jnp.sum(tw)
            cnt_ref[b, 2] += jnp.sum(km)

    @pl.when((ph == 1) & (i == 0))
    def _prologue():
        cols = vec_ref[...] * inv_n
        cnorm = jnp.sqrt(jnp.sum(cols * cols, axis=0, keepdims=True)) + 1e-8
        cols_ref[...] = cols / cnorm
        dq = dq_ref[...]
        dnorm = jnp.sqrt(jnp.sum(dq * dq, axis=1, keepdims=True)) + 1e-8
        proto_ref[...] = dq / dnorm
        for j in range(2 * batch):
            acc_ref[j] = 0.0

    @pl.when(ph == 1)
    def _pass2():
        ka0 = cnt_ref[0, 2] > 0.0
        ka1 = cnt_ref[1, 2] > 0.0
        for b in range(batch):
            _, _, tw = _pred_masks(no_ref[b], tg_ref[b])
            f = f_ref[b]
            sq = jnp.sum(f * f, axis=0, keepdims=True)
            rn = 1.0 / (jnp.sqrt(sq) + 1e-8)
            dots = jnp.dot(proto_ref[...], f,
                           preferred_element_type=jnp.float32)
            ek = jnp.sum(jnp.exp(dots * rn) * tw)
            ckv = jax.lax.dot_general(
                cols_ref[:, 0:batch], f, (((0,), (0,)), ((), ())),
                preferred_element_type=jnp.float32)
            ekv = jnp.exp(ckv * rn) * tw
            ek = ek + jnp.where(ka0, jnp.sum(ekv[0:1, :]), 0.0)
            if b >= 1:
                ek = ek + jnp.where(ka1, jnp.sum(ekv[1:2, :]), 0.0)
            sv = jnp.sum(f * cols_ref[:, batch + b:batch + b + 1],
                         axis=0, keepdims=True)
            s_b = jnp.sum(sv * rn * tw)
            acc_ref[b] += s_b
            acc_ref[batch + b] += ek

    @pl.when((ph == 1) & (i == nblocks - 1))
    def _epilogue():
        et = jnp.float32(0.0)
        ek = jnp.float32(0.0)
        any_c = False
        for b in range(batch):
            c_b = (cnt_ref[b, 0] > 0.0) & (cnt_ref[b, 1] > 0.0)
            et = et + jnp.where(c_b, jnp.exp(acc_ref[b]), 0.0)
            ek = ek + jnp.where(c_b, acc_ref[batch + b], 0.0)
            any_c = c_b | any_c
        denom = jnp.where(any_c, ek, 1.0)
        loss = jnp.where(any_c, (-1.0 / batch) * jnp.log(et / denom), 0.0)
        out_ref[0, 0] = loss


@jax.jit
def _run(net_output, feature, target, kidney_deque):
    b, c, d, h, w = net_output.shape
    fd = feature.shape[1]
    q = kidney_deque.shape[0]
    n_total = d * h * w
    nb = 16384
    while n_total % nb != 0:
        nb //= 2
    nblocks = n_total // nb

    no = net_output.reshape(b, c, n_total)
    f = feature.reshape(b, fd, n_total)
    tg = target.reshape(b, 1, n_total)

    loss = pl.pallas_call(
        functools.partial(_fused_kernel, batch=b, n_total=n_total, q=q),
        grid=(2, nblocks),
        in_specs=[
            pl.BlockSpec((b, c, nb), lambda p, i: (0, 0, i)),
            pl.BlockSpec((b, 1, nb), lambda p, i: (0, 0, i)),
            pl.BlockSpec((b, fd, nb), lambda p, i: (0, 0, i)),
            pl.BlockSpec((q, fd), lambda p, i: (0, 0)),
        ],
        out_specs=pl.BlockSpec(memory_space=pltpu.SMEM),
        out_shape=jax.ShapeDtypeStruct((1, 1), jnp.float32),
        scratch_shapes=[
            pltpu.VMEM((fd, 2 * b * 2), jnp.float32),
            pltpu.SMEM((b, 3), jnp.float32),
            pltpu.VMEM((q, fd), jnp.float32),
            pltpu.VMEM((fd, 2 * b * 2), jnp.float32),
            pltpu.SMEM((2 * b,), jnp.float32),
        ],
    )(no, tg, f, kidney_deque)

    return loss[0, 0]


def kernel(net_output, feature, target, kidney_deque, background_deque):
    del background_deque
    return _run(net_output, feature, target, kidney_deque)
